# Initial kernel scaffold; baseline (speedup 1.0000x reference)
import jax
import jax.numpy as jnp
from jax.experimental import pallas as pl


def kernel(x, pos, edge_attr, W_in, b_in, W_na, b_na, W1_0, b1_0, W2_0, b2_0, Wsc_0, bsc_0, W1_1, b1_1, W2_1, b2_1, Wsc_1, bsc_1, Wp1, bp1, Wp2, bp2, Wn1, bn1, Wn2, bn2, edge_index, batch):
    raise NotImplementedError("write your pallas kernel here")


# speedup vs baseline: 2.5420x; 2.5420x over previous
import functools

import jax
import jax.numpy as jnp
from jax import lax
from jax.experimental import pallas as pl
from jax.experimental.pallas import tpu as pltpu
from jax.experimental.pallas import tpu_sc as plsc

N = 10000
E = 320000
D = 128
G = 64
AVG_NB = 10.0

NCORE = 2
NSUB = 16
NW = NCORE * NSUB
EPT = E // NW
CH = 80
NCH = EPT // CH
NP = 10240
ROWS_T = NP // NSUB
ZR = 128
NZ = ROWS_T // ZR

_HI = lax.Precision.HIGHEST


def _dot(a, b):
    return jnp.dot(a, b, precision=_HI, preferred_element_type=jnp.float32)


def _geom_body(px_hbm, py_hbm, pz_hbm, src_hbm, dst_hbm, out_hbm,
               pxv, pyv, pzv, idx_s, idx_d, ob):
    cid = lax.axis_index("c")
    sid = lax.axis_index("s")
    wid = sid * NCORE + cid
    ebase = wid * EPT

    pltpu.sync_copy(px_hbm, pxv)
    pltpu.sync_copy(py_hbm, pyv)
    pltpu.sync_copy(pz_hbm, pzv)

    def zb(i, _):
        ob[pl.ds(i * 16, 16)] = jnp.zeros((16,), jnp.float32)
        return 0
    lax.fori_loop(0, CH * 8 // 16, zb, 0)

    lanes = jnp.arange(16, dtype=jnp.int32)

    def chunk(i, _):
        e0 = ebase + i * CH
        pltpu.sync_copy(src_hbm.at[pl.ds(e0, CH)], idx_s)
        pltpu.sync_copy(dst_hbm.at[pl.ds(e0, CH)], idx_d)
        for j in range(CH // 16):
            es = idx_s[pl.ds(16 * j, 16)]
            ed = idx_d[pl.ds(16 * j, 16)]
            dx = plsc.load_gather(pxv, [es]) - plsc.load_gather(pxv, [ed])
            dy = plsc.load_gather(pyv, [es]) - plsc.load_gather(pyv, [ed])
            dz = plsc.load_gather(pzv, [es]) - plsc.load_gather(pzv, [ed])
            ss = dx * dx + dy * dy + dz * dz
            rows = (lanes + 16 * j) * 8
            plsc.store_scatter(ob, [rows], dx)
            plsc.store_scatter(ob, [rows + 1], dy)
            plsc.store_scatter(ob, [rows + 2], dz)
            plsc.store_scatter(ob, [rows + 3], ss)
        pltpu.sync_copy(ob, out_hbm.at[pl.ds(e0 * 8, CH * 8)])
        return 0

    lax.fori_loop(0, NCH, chunk, 0)


_geom_kernel = functools.partial(
    pl.kernel,
    out_type=jax.ShapeDtypeStruct((E * 8,), jnp.float32),
    mesh=plsc.VectorSubcoreMesh(core_axis_name="c", subcore_axis_name="s"),
    scratch_types=[
        pltpu.VMEM((N,), jnp.float32),
        pltpu.VMEM((N,), jnp.float32),
        pltpu.VMEM((N,), jnp.float32),
        pltpu.VMEM((CH,), jnp.int32),
        pltpu.VMEM((CH,), jnp.int32),
        pltpu.VMEM((CH * 8,), jnp.float32),
    ],
    compiler_params=pltpu.CompilerParams(needs_layout_passes=False),
)(_geom_body)


def _edge_body(a_hbm, q_hbm, src_hbm, dst_hbm, out_hbm,
               idx_s, idx_d, abuf, qbuf, zbuf, agg, sem):
    cid = lax.axis_index("c")
    sid = lax.axis_index("s")
    wid = sid * NCORE + cid
    ebase = wid * EPT
    row0 = sid * ROWS_T

    def zb(i, _):
        for k in range(8):
            zbuf[i, pl.ds(16 * k, 16)] = jnp.zeros((16,), jnp.float32)
        return 0
    lax.fori_loop(0, ZR, zb, 0)
    for j in range(NZ):
        pltpu.sync_copy(zbuf, agg.at[pl.ds(row0 + j * ZR, ZR)])
    plsc.subcore_barrier()

    def chunk(i, _):
        e0 = ebase + i * CH
        pltpu.sync_copy(src_hbm.at[pl.ds(e0, CH)], idx_s)
        pltpu.sync_copy(dst_hbm.at[pl.ds(e0, CH)], idx_d)
        pltpu.async_copy(a_hbm.at[idx_s], abuf, sem).wait()
        pltpu.sync_copy(q_hbm.at[pl.ds(e0, CH)], qbuf)

        def erow(e, _):
            for k in range(8):
                z = abuf[e, pl.ds(16 * k, 16)] + qbuf[e, pl.ds(16 * k, 16)]
                abuf[e, pl.ds(16 * k, 16)] = z / (1.0 + jnp.exp(-z))
            return 0
        lax.fori_loop(0, CH, erow, 0)

        pltpu.sync_copy(abuf, agg.at[idx_d], add=True)
        return 0

    lax.fori_loop(0, NCH, chunk, 0)
    plsc.subcore_barrier()

    for j in range(NZ):
        pltpu.sync_copy(agg.at[pl.ds(row0 + j * ZR, ZR)], zbuf)
        pltpu.sync_copy(zbuf, out_hbm.at[cid, pl.ds(row0 + j * ZR, ZR)])


_edge_kernel = functools.partial(
    pl.kernel,
    out_type=jax.ShapeDtypeStruct((NCORE, NP, D), jnp.float32),
    mesh=plsc.VectorSubcoreMesh(core_axis_name="c", subcore_axis_name="s"),
    scratch_types=[
        pltpu.VMEM((CH,), jnp.int32),
        pltpu.VMEM((CH,), jnp.int32),
        pltpu.VMEM((CH, D), jnp.float32),
        pltpu.VMEM((CH, D), jnp.float32),
        pltpu.VMEM((ZR, D), jnp.float32),
        pltpu.VMEM_SHARED((NP, D), jnp.float32),
        pltpu.SemaphoreType.DMA,
    ],
    compiler_params=pltpu.CompilerParams(needs_layout_passes=False),
)(_edge_body)


BN = 1000
GN = N // BN
BE = 1280
GE = E // BE

_full = lambda s: pl.BlockSpec(s, lambda i: (0,) * len(s))


def _pre_body(x_ref, pos_ref, Win, bin_, Wna, bna, W1n0, Wp0, na_ref, a0_ref):
    xb = x_ref[...]
    na_ref[...] = _dot(xb, Wna[...]) + bna[...]
    nf0 = _dot(xb, Win[...]) + bin_[...]
    a0_ref[...] = _dot(nf0, W1n0[...]) + _dot(pos_ref[...], Wp0[...])


def _tc_pre(x, pos8, Win, bin_, Wna, bna, W1n0, Wp0):
    return pl.pallas_call(
        _pre_body,
        grid=(GN,),
        in_specs=[
            pl.BlockSpec((BN, D), lambda i: (i, 0)),
            pl.BlockSpec((BN, 8), lambda i: (i, 0)),
            _full((D, D)), _full((1, D)), _full((D, D)), _full((1, D)),
            _full((D, D)), _full((8, D)),
        ],
        out_specs=[
            pl.BlockSpec((BN, D), lambda i: (i, 0)),
            pl.BlockSpec((BN, D), lambda i: (i, 0)),
        ],
        out_shape=[
            jax.ShapeDtypeStruct((N, D), jnp.float32),
            jax.ShapeDtypeStruct((N, D), jnp.float32),
        ],
    )(x, pos8, Win, bin_, Wna, bna, W1n0, Wp0)


def _q_body(ea_ref, gm_ref, Wea0, Wg0, wr0, b10, Wea1, Wg1, wr1, b11,
            q0_ref, q1_ref):
    ab = ea_ref[...]
    gb = gm_ref[...]
    r = jnp.sqrt(gb[:, 3:4] + 1e-12)
    q0_ref[...] = _dot(ab, Wea0[...]) + _dot(gb, Wg0[...]) + r * wr0[...] + b10[...]
    q1_ref[...] = _dot(ab, Wea1[...]) + _dot(gb, Wg1[...]) + r * wr1[...] + b11[...]


def _tc_q(ea, gm, Wea0, Wg0, wr0, b10, Wea1, Wg1, wr1, b11):
    return pl.pallas_call(
        _q_body,
        grid=(GE,),
        in_specs=[
            pl.BlockSpec((BE, 16), lambda i: (i, 0)),
            pl.BlockSpec((BE, 8), lambda i: (i, 0)),
            _full((16, D)), _full((8, D)), _full((1, D)), _full((1, D)),
            _full((16, D)), _full((8, D)), _full((1, D)), _full((1, D)),
        ],
        out_specs=[
            pl.BlockSpec((BE, D), lambda i: (i, 0)),
            pl.BlockSpec((BE, D), lambda i: (i, 0)),
        ],
        out_shape=[
            jax.ShapeDtypeStruct((E, D), jnp.float32),
            jax.ShapeDtypeStruct((E, D), jnp.float32),
        ],
    )(ea, gm, Wea0, Wg0, wr0, b10, Wea1, Wg1, wr1, b11)


def _mid_body(p0_ref, p1_ref, na_ref, W2, Wsc, bb, W1n1, a1_ref):
    agg = (p0_ref[...] + p1_ref[...]) * (1.0 / AVG_NB)
    nf1 = _dot(agg, W2[...]) + _dot(na_ref[...], Wsc[...]) + bb[...]
    a1_ref[...] = _dot(nf1, W1n1[...])


def _tc_mid(p0, p1, na, W2, Wsc, bb, W1n1):
    return pl.pallas_call(
        _mid_body,
        grid=(GN,),
        in_specs=[
            pl.BlockSpec((BN, D), lambda i: (i, 0)),
            pl.BlockSpec((BN, D), lambda i: (i, 0)),
            pl.BlockSpec((BN, D), lambda i: (i, 0)),
            _full((D, D)), _full((D, D)), _full((1, D)), _full((D, D)),
        ],
        out_specs=pl.BlockSpec((BN, D), lambda i: (i, 0)),
        out_shape=jax.ShapeDtypeStruct((N, D), jnp.float32),
    )(p0, p1, na, W2, Wsc, bb, W1n1)


def _silu(v):
    return v * jax.nn.sigmoid(v)


def _fin_body(p0_ref, p1_ref, na_ref, bt_ref, W2, Wsc, bb,
              Wn1, bn1, Wn2p, bn2p, Wp1, bp1, Wp2p, bp2p,
              nlp_ref, gh_ref, lgp_ref, acc):
    i = pl.program_id(0)
    agg = (p0_ref[...] + p1_ref[...]) * (1.0 / AVG_NB)
    nf2 = _dot(agg, W2[...]) + _dot(na_ref[...], Wsc[...]) + bb[...]
    h = _silu(_dot(nf2, Wn1[...]) + bn1[...])
    nlp_ref[...] = _dot(h, Wn2p[...]) + bn2p[...]
    oh = (bt_ref[...] == lax.broadcasted_iota(jnp.int32, (BN, G), 1))
    oh = oh.astype(jnp.float32)
    part = lax.dot_general(oh, nf2, (((0,), (0,)), ((), ())),
                           precision=_HI, preferred_element_type=jnp.float32)

    @pl.when(i == 0)
    def _():
        acc[...] = part

    @pl.when(i > 0)
    def _():
        acc[...] = acc[...] + part

    @pl.when(i == GN - 1)
    def _():
        gh = acc[...]
        gh_ref[...] = gh
        hg = _silu(_dot(gh, Wp1[...]) + bp1[...])
        lgp_ref[...] = _dot(hg, Wp2p[...]) + bp2p[...]


def _tc_fin(p0, p1, na, bt, W2, Wsc, bb, Wn1, bn1, Wn2p, bn2p,
            Wp1, bp1, Wp2p, bp2p):
    return pl.pallas_call(
        _fin_body,
        grid=(GN,),
        in_specs=[
            pl.BlockSpec((BN, D), lambda i: (i, 0)),
            pl.BlockSpec((BN, D), lambda i: (i, 0)),
            pl.BlockSpec((BN, D), lambda i: (i, 0)),
            pl.BlockSpec((BN, 1), lambda i: (i, 0)),
            _full((D, D)), _full((D, D)), _full((1, D)),
            _full((D, G)), _full((1, G)), _full((G, D)), _full((1, D)),
            _full((D, G)), _full((1, G)), _full((G, D)), _full((1, D)),
        ],
        out_specs=[
            pl.BlockSpec((BN, D), lambda i: (i, 0)),
            _full((G, D)),
            _full((G, D)),
        ],
        out_shape=[
            jax.ShapeDtypeStruct((N, D), jnp.float32),
            jax.ShapeDtypeStruct((G, D), jnp.float32),
            jax.ShapeDtypeStruct((G, D), jnp.float32),
        ],
        scratch_shapes=[pltpu.VMEM((G, D), jnp.float32)],
    )(p0, p1, na, bt, W2, Wsc, bb, Wn1, bn1, Wn2p, bn2p,
      Wp1, bp1, Wp2p, bp2p)


def kernel(x, pos, edge_attr, W_in, b_in, W_na, b_na,
           W1_0, b1_0, W2_0, b2_0, Wsc_0, bsc_0,
           W1_1, b1_1, W2_1, b2_1, Wsc_1, bsc_1,
           Wp1, bp1, Wp2, bp2, Wn1, bn1, Wn2, bn2,
           edge_index, batch):
    f32 = jnp.float32
    src = edge_index[0]
    dst = edge_index[1]
    row = lambda v: v.reshape(1, -1)

    pos8 = jnp.pad(pos, ((0, 0), (0, 5)))
    px = jnp.asarray(pos[:, 0])
    py = jnp.asarray(pos[:, 1])
    pz = jnp.asarray(pos[:, 2])

    W1n0 = W1_0[:D]
    Wp0 = jnp.pad(W1_0[D:D + 3], ((0, 5), (0, 0)))
    Wea0 = W1_0[D + 3:D + 19]
    Wg0 = jnp.pad(W1_0[D + 19:D + 22], ((0, 5), (0, 0)))
    wr0 = row(W1_0[D + 22])
    W1n1 = W1_1[:D]
    Wea1 = W1_1[D:D + 16]
    Wg1 = jnp.pad(W1_1[D + 16:D + 19], ((0, 5), (0, 0)))
    wr1 = row(W1_1[D + 19])

    bb0 = row(b2_0 + bsc_0)
    bb1 = row(b2_1 + bsc_1)
    Wn2p = jnp.pad(Wn2, ((0, 0), (0, D - 3)))
    bn2p = row(jnp.pad(bn2, (0, D - 3)))
    Wp2p = jnp.pad(Wp2, ((0, 0), (0, D - 1)))
    bp2p = row(jnp.pad(bp2, (0, D - 1)))

    na, A0 = _tc_pre(x, pos8, W_in, row(b_in), W_na, row(b_na), W1n0, Wp0)

    geom = _geom_kernel(px, py, pz, src, dst).reshape(E, 8)

    Q0, Q1 = _tc_q(edge_attr, geom, Wea0, Wg0, wr0, row(b1_0),
                   Wea1, Wg1, wr1, row(b1_1))

    p = _edge_kernel(A0, Q0, src, dst)
    A1 = _tc_mid(p[0, :N], p[1, :N], na, W2_0, Wsc_0, bb0, W1n1)

    p = _edge_kernel(A1, Q1, src, dst)
    nlp, g_h, lgp = _tc_fin(p[0, :N], p[1, :N], na, batch.reshape(N, 1).astype(jnp.int32),
                            W2_1, Wsc_1, bb1, Wn1, row(bn1), Wn2p, bn2p,
                            Wp1, row(bp1), Wp2p, bp2p)

    return (lgp[:, :1], nlp[:, :3], g_h)
